# baseline (speedup 1.0000x reference)
import jax
import jax.numpy as jnp
from jax.experimental import pallas as pl
from jax.experimental.pallas import tpu as pltpu

_B, _T, _P, _H = 8, 4, 1025, 1280
_ROWS = 9
_PH = _P * _H
_N = _B * _T
_NCHUNK = 5
_CHUNK = _PH // _NCHUNK


def _body(ids_ref, gate_ref, hid_ref, emb_ref, tile_hbm, out_ref, scr, sems):
    b = pl.program_id(0)
    t = pl.program_id(1)
    s = b * _T + t

    def copies(s2, slot):
        row = ids_ref[s2 // _T]
        c0 = (s2 % _T) * _PH
        return [
            pltpu.make_async_copy(
                tile_hbm.at[row, pl.ds(c0 + k * _CHUNK, _CHUNK)],
                scr.at[slot, pl.ds(k * _CHUNK, _CHUNK)],
                sems.at[slot, k],
            )
            for k in range(_NCHUNK)
        ]

    def start(s2, slot):
        for c in copies(s2, slot):
            c.start()

    @pl.when(s == 0)
    def _():
        start(0, 0)

    @pl.when(s < _N - 1)
    def _():
        start(s + 1, (s + 1) % 2)

    slot = s % 2
    for c in copies(s, slot):
        c.wait()

    tg = jnp.tanh(gate_ref[0])
    tile2 = scr[slot].reshape(_P, _H)
    out_ref[0, 0] = (hid_ref[0, 0] + (1.0 - tg) * emb_ref[...]) + tg * tile2


def kernel(hidden_state, aspect_ratio_ids, gate, embedding, tile_table):
    ids = aspect_ratio_ids.astype(jnp.int32)
    kfn = pl.pallas_call(
        _body,
        grid_spec=pltpu.PrefetchScalarGridSpec(
            num_scalar_prefetch=1,
            grid=(_B, _T),
            in_specs=[
                pl.BlockSpec(memory_space=pltpu.SMEM),
                pl.BlockSpec((1, 1, _P, _H), lambda b, t, ids: (b, t, 0, 0)),
                pl.BlockSpec((_P, _H), lambda b, t, ids: (0, 0)),
                pl.BlockSpec(memory_space=pl.ANY),
            ],
            out_specs=pl.BlockSpec((1, 1, _P, _H), lambda b, t, ids: (b, t, 0, 0)),
            scratch_shapes=[
                pltpu.VMEM((2, _PH), jnp.float32),
                pltpu.SemaphoreType.DMA((2, _NCHUNK)),
            ],
        ),
        out_shape=jax.ShapeDtypeStruct(hidden_state.shape, hidden_state.dtype),
        compiler_params=pltpu.CompilerParams(
            dimension_semantics=("arbitrary", "arbitrary"),
            vmem_limit_bytes=100 * 1024 * 1024,
        ),
    )
    return kfn(ids, gate, hidden_state, embedding, tile_table)


# speedup vs baseline: 1.0075x; 1.0075x over previous
import jax
import jax.numpy as jnp
from jax.experimental import pallas as pl
from jax.experimental.pallas import tpu as pltpu

_B, _T, _P, _H = 8, 4, 1025, 1280
_ROWS = 9
_PH = _P * _H
_N = _B * _T


def _body(ids_ref, gate_ref, hid_ref, emb_hbm, tile_hbm, out_ref,
          scr, emb_scr, sems, emb_sem):
    b = pl.program_id(0)
    t = pl.program_id(1)
    s = b * _T + t

    def start(s2, slot):
        row = ids_ref[s2 // _T]
        c0 = (s2 % _T) * _PH
        pltpu.make_async_copy(
            tile_hbm.at[row, pl.ds(c0, _PH)], scr.at[slot], sems.at[slot]
        ).start()

    @pl.when(s == 0)
    def _():
        pltpu.make_async_copy(emb_hbm, emb_scr, emb_sem).start()
        start(0, 0)
        pltpu.make_async_copy(emb_hbm, emb_scr, emb_sem).wait()

    @pl.when(s < _N - 1)
    def _():
        start(s + 1, (s + 1) % 2)

    slot = s % 2
    row = ids_ref[b]
    c0 = t * _PH
    pltpu.make_async_copy(
        tile_hbm.at[row, pl.ds(c0, _PH)], scr.at[slot], sems.at[slot]
    ).wait()

    tg = jnp.tanh(gate_ref[0])
    tile2 = scr[slot].reshape(_P, _H)
    out_ref[0, 0] = (hid_ref[0, 0] + (1.0 - tg) * emb_scr[...]) + tg * tile2


def kernel(hidden_state, aspect_ratio_ids, gate, embedding, tile_table):
    ids = aspect_ratio_ids.astype(jnp.int32)
    kfn = pl.pallas_call(
        _body,
        grid_spec=pltpu.PrefetchScalarGridSpec(
            num_scalar_prefetch=1,
            grid=(_B, _T),
            in_specs=[
                pl.BlockSpec(memory_space=pltpu.SMEM),
                pl.BlockSpec((1, 1, _P, _H), lambda b, t, ids: (b, t, 0, 0)),
                pl.BlockSpec(memory_space=pl.ANY),
                pl.BlockSpec(memory_space=pl.ANY),
            ],
            out_specs=pl.BlockSpec((1, 1, _P, _H), lambda b, t, ids: (b, t, 0, 0)),
            scratch_shapes=[
                pltpu.VMEM((2, _PH), jnp.float32),
                pltpu.VMEM((_P, _H), jnp.float32),
                pltpu.SemaphoreType.DMA((2,)),
                pltpu.SemaphoreType.DMA,
            ],
        ),
        out_shape=jax.ShapeDtypeStruct(hidden_state.shape, hidden_state.dtype),
        compiler_params=pltpu.CompilerParams(
            dimension_semantics=("arbitrary", "arbitrary"),
            vmem_limit_bytes=100 * 1024 * 1024,
        ),
    )
    return kfn(ids, gate, hidden_state, embedding, tile_table)
